# Initial kernel scaffold; baseline (speedup 1.0000x reference)
import jax
import jax.numpy as jnp
from jax.experimental import pallas as pl


def kernel(seq, target, embed_weight):
    raise NotImplementedError("write your pallas kernel here")


# speedup vs baseline: 9.8309x; 9.8309x over previous
import functools

import jax
import jax.numpy as jnp
from jax import lax
from jax.experimental import pallas as pl
from jax.experimental.pallas import tpu as pltpu
from jax.experimental.pallas import tpu_sc as plsc

D = 64
B = 4096
HIST = 50
NC, NS, L = 2, 16, 16
NW = NC * NS
BPW = B // NW
GROUP = 2 * HIST
GROUPS = BPW // 2
NBUF = 8

_mesh = plsc.VectorSubcoreMesh(core_axis_name="c", subcore_axis_name="s")


@functools.partial(
    pl.kernel,
    mesh=_mesh,
    out_type=jax.ShapeDtypeStruct((B,), jnp.float32),
    scratch_types=(
        [
            pltpu.VMEM((GROUPS, GROUP), jnp.int32),
            pltpu.VMEM((BPW,), jnp.int32),
            pltpu.VMEM((BPW, D), jnp.float32),
            pltpu.VMEM((BPW,), jnp.float32),
        ]
        + [pltpu.VMEM((GROUP, D), jnp.float32) for _ in range(NBUF)]
        + [pltpu.SemaphoreType.DMA for _ in range(NBUF + 1)]
    ),
    compiler_params=pltpu.CompilerParams(use_tc_tiling_on_sc=False),
)
def _bprmf_sc(seq_hbm, tgt_hbm, table_hbm, out_hbm,
              idx_v, tgt_idx, tgt_rows, out_buf, *rest):
    bufs = rest[:NBUF]
    sems = rest[NBUF:2 * NBUF]
    tsem = rest[2 * NBUF]

    wid = lax.axis_index("s") * NC + lax.axis_index("c")
    base = wid * BPW

    pltpu.sync_copy(seq_hbm.at[wid], idx_v)
    pltpu.sync_copy(tgt_hbm.at[wid], tgt_idx)

    pltpu.async_copy(table_hbm.at[tgt_idx], tgt_rows, tsem)

    for b_ in range(NBUF):
        pltpu.async_copy(table_hbm.at[idx_v.at[b_]], bufs[b_], sems[b_])

    pltpu.make_async_copy(table_hbm.at[tgt_idx], tgt_rows, tsem).wait()

    lane = lax.iota(jnp.int32, L)
    zero = jnp.zeros((L,), jnp.float32)
    one = jnp.ones((L,), jnp.float32)

    def _allreduce_sum(v):
        for k in (8, 4, 2, 1):
            v = v + v.at[lane ^ k].get(mode="promise_in_bounds")
        return v

    def _process(gg, buf, res, pos0):
        for r in range(2):
            rowbase = r * HIST
            pos = pos0 + r

            def jbody(j, accs):
                a0, a1, a2, a3 = accs
                row = rowbase + j
                a0 = a0 + buf[row, pl.ds(0, L)]
                a1 = a1 + buf[row, pl.ds(L, L)]
                a2 = a2 + buf[row, pl.ds(2 * L, L)]
                a3 = a3 + buf[row, pl.ds(3 * L, L)]
                return (a0, a1, a2, a3)

            a0, a1, a2, a3 = lax.fori_loop(0, HIST, jbody,
                                           (zero, zero, zero, zero))

            rr = 2 * gg + r
            t0 = tgt_rows[rr, pl.ds(0, L)]
            t1 = tgt_rows[rr, pl.ds(L, L)]
            t2 = tgt_rows[rr, pl.ds(2 * L, L)]
            t3 = tgt_rows[rr, pl.ds(3 * L, L)]
            dotv = a0 * t0 + a1 * t1 + a2 * t2 + a3 * t3

            s0 = idx_v[gg, pl.ds(rowbase, L)]
            s1 = idx_v[gg, pl.ds(rowbase + L, L)]
            s2 = idx_v[gg, pl.ds(rowbase + 2 * L, L)]
            s3 = idx_v[gg, pl.ds(rowbase + GROUP // 2 - L, L)]
            w = (jnp.where(s0 != 0, one, zero)
                 + jnp.where(s1 != 0, one, zero)
                 + jnp.where(s2 != 0, one, zero)
                 + jnp.where((lane >= L - 2) & (s3 != 0), one, zero))
            pred_v = _allreduce_sum(dotv) / _allreduce_sum(w)
            res = jnp.where(lane == pos, pred_v, res)
        return res

    def outer(i, carry):
        res = zero
        for b_ in range(NBUF):
            gg = i * NBUF + b_
            pltpu.make_async_copy(table_hbm.at[idx_v.at[b_]],
                                  bufs[b_], sems[b_]).wait()
            res = _process(gg, bufs[b_], res, 2 * b_)

            @pl.when(gg + NBUF < GROUPS)
            def _():
                pltpu.async_copy(table_hbm.at[idx_v.at[gg + NBUF]],
                                 bufs[b_], sems[b_])
        out_buf[pl.ds(i * L, L)] = res
        return carry

    lax.fori_loop(0, GROUPS // NBUF, outer, 0)

    pltpu.sync_copy(out_buf, out_hbm.at[pl.ds(base, BPW)])


def kernel(seq, target, embed_weight):
    seq_w = seq.astype(jnp.int32).reshape(NW, GROUPS, GROUP)
    tgt_w = target.astype(jnp.int32).reshape(NW, BPW)
    return _bprmf_sc(seq_w, tgt_w, embed_weight)
